# Initial kernel scaffold; baseline (speedup 1.0000x reference)
import jax
import jax.numpy as jnp
from jax.experimental import pallas as pl


def kernel(annotation, edge_index, etypes, W_e, b_e, w_ih, w_hh, b_ih, b_hh, w_out, b_out):
    raise NotImplementedError("write your pallas kernel here")


# speedup vs baseline: 17.1889x; 17.1889x over previous
import functools

import jax
import jax.numpy as jnp
from jax import lax
from jax.experimental import pallas as pl
from jax.experimental.pallas import tpu as pltpu
from jax.experimental.pallas import tpu_sc as plsc

N = 10000
E = 320000
ANN = 64
OUT = 128
N_STEPS = 5
N_ETYPES = 4

NSC = 2
NTILES = 16
ROWS_A = 624
ROWS_TAIL = N - NTILES * ROWS_A
E_PER_TILE = E // (NSC * NTILES)
K = 80
N_CHUNKS = E_PER_TILE // K

BM = 2000


def _transform_body(h_ref, w_ref, b_ref, t_ref):
    t_ref[...] = jnp.dot(h_ref[...], w_ref[...],
                         preferred_element_type=jnp.float32) + b_ref[...]


def _transform(h, W_all, b_all):
    return pl.pallas_call(
        _transform_body,
        grid=(N // BM,),
        in_specs=[pl.BlockSpec((BM, OUT), lambda i: (i, 0)),
                  pl.BlockSpec((OUT, 4 * OUT), lambda i: (0, 0)),
                  pl.BlockSpec((1, 4 * OUT), lambda i: (0, 0))],
        out_specs=pl.BlockSpec((BM, 4 * OUT), lambda i: (i, 0)),
        out_shape=jax.ShapeDtypeStruct((N, 4 * OUT), jnp.float32),
    )(h, W_all, b_all)


def _gru_math(a0_ref, a1_ref, h_ref, wih_ref, whh_ref, bih_ref, bhh_ref):
    a = a0_ref[...] + a1_ref[...]
    h = h_ref[...]
    gi = jnp.dot(a, wih_ref[...], preferred_element_type=jnp.float32) + bih_ref[...]
    gh = jnp.dot(h, whh_ref[...], preferred_element_type=jnp.float32) + bhh_ref[...]
    r = jax.nn.sigmoid(gi[:, :OUT] + gh[:, :OUT])
    z = jax.nn.sigmoid(gi[:, OUT:2 * OUT] + gh[:, OUT:2 * OUT])
    n = jnp.tanh(gi[:, 2 * OUT:] + r * gh[:, 2 * OUT:])
    return (1.0 - z) * n + z * h


def _gru_tr_body(a0_ref, a1_ref, h_ref, wih_ref, whh_ref, bih_ref, bhh_ref,
                 wall_ref, ball_ref, h_out_ref, t_out_ref):
    hn = _gru_math(a0_ref, a1_ref, h_ref, wih_ref, whh_ref, bih_ref, bhh_ref)
    h_out_ref[...] = hn
    t_out_ref[...] = jnp.dot(hn, wall_ref[...],
                             preferred_element_type=jnp.float32) + ball_ref[...]


def _gru_transform(a0, a1, h, wihT, whhT, bih, bhh, W_all, b_all):
    return pl.pallas_call(
        _gru_tr_body,
        grid=(N // BM,),
        in_specs=[pl.BlockSpec((BM, OUT), lambda i: (i, 0)),
                  pl.BlockSpec((BM, OUT), lambda i: (i, 0)),
                  pl.BlockSpec((BM, OUT), lambda i: (i, 0)),
                  pl.BlockSpec((OUT, 3 * OUT), lambda i: (0, 0)),
                  pl.BlockSpec((OUT, 3 * OUT), lambda i: (0, 0)),
                  pl.BlockSpec((1, 3 * OUT), lambda i: (0, 0)),
                  pl.BlockSpec((1, 3 * OUT), lambda i: (0, 0)),
                  pl.BlockSpec((OUT, 4 * OUT), lambda i: (0, 0)),
                  pl.BlockSpec((1, 4 * OUT), lambda i: (0, 0))],
        out_specs=[pl.BlockSpec((BM, OUT), lambda i: (i, 0)),
                   pl.BlockSpec((BM, 4 * OUT), lambda i: (i, 0))],
        out_shape=[jax.ShapeDtypeStruct((N, OUT), jnp.float32),
                   jax.ShapeDtypeStruct((N, 4 * OUT), jnp.float32)],
    )(a0, a1, h, wihT, whhT, bih, bhh, W_all, b_all)


def _gru_fin_body(a0_ref, a1_ref, h_ref, wih_ref, whh_ref, bih_ref, bhh_ref,
                  ann_ref, wh_ref, wa_ref, bo_ref, l_out_ref):
    hn = _gru_math(a0_ref, a1_ref, h_ref, wih_ref, whh_ref, bih_ref, bhh_ref)
    logit = (jnp.sum(hn * wh_ref[...], axis=1, keepdims=True)
             + jnp.sum(ann_ref[...] * wa_ref[...], axis=1, keepdims=True)
             + bo_ref[...])
    l_out_ref[...] = logit


def _gru_final(a0, a1, h, wihT, whhT, bih, bhh, ann, w_h, w_a, b_o):
    return pl.pallas_call(
        _gru_fin_body,
        grid=(N // BM,),
        in_specs=[pl.BlockSpec((BM, OUT), lambda i: (i, 0)),
                  pl.BlockSpec((BM, OUT), lambda i: (i, 0)),
                  pl.BlockSpec((BM, OUT), lambda i: (i, 0)),
                  pl.BlockSpec((OUT, 3 * OUT), lambda i: (0, 0)),
                  pl.BlockSpec((OUT, 3 * OUT), lambda i: (0, 0)),
                  pl.BlockSpec((1, 3 * OUT), lambda i: (0, 0)),
                  pl.BlockSpec((1, 3 * OUT), lambda i: (0, 0)),
                  pl.BlockSpec((BM, ANN), lambda i: (i, 0)),
                  pl.BlockSpec((1, OUT), lambda i: (0, 0)),
                  pl.BlockSpec((1, ANN), lambda i: (0, 0)),
                  pl.BlockSpec((1, 1), lambda i: (0, 0))],
        out_specs=pl.BlockSpec((BM, 1), lambda i: (i, 0)),
        out_shape=jax.ShapeDtypeStruct((N, 1), jnp.float32),
    )(a0, a1, h, wihT, whhT, bih, bhh, ann, w_h, w_a, b_o)


NPAD = 10240


def _argmax_body(l_ref, p_ref):
    row = l_ref[...]
    m = jnp.max(row)
    ii = lax.broadcasted_iota(jnp.int32, row.shape, 1)
    cand = jnp.where(row == m, ii, jnp.int32(NPAD))
    p_ref[...] = jnp.min(cand)[None, None]


def _argmax(logits_padded):
    return pl.pallas_call(
        _argmax_body,
        out_shape=jax.ShapeDtypeStruct((1, 1), jnp.int32),
    )(logits_padded)


def _sc_scatter(table, fidx, dst, zeros):
    mesh = plsc.VectorSubcoreMesh(core_axis_name="c", subcore_axis_name="s",
                                  num_cores=NSC, num_subcores=NTILES)

    @functools.partial(
        pl.kernel,
        out_type=jax.ShapeDtypeStruct((NSC, N, OUT), jnp.float32),
        mesh=mesh,
        scratch_types=[
            pltpu.VMEM((K,), jnp.int32),
            pltpu.VMEM((K,), jnp.int32),
            pltpu.VMEM((K, OUT), jnp.float32),
            pltpu.VMEM_SHARED((N, OUT), jnp.float32),
            pltpu.SemaphoreType.DMA,
        ],
    )
    def k(table_hbm, fidx_hbm, dst_hbm, zeros_hbm, out_hbm,
          idx_v, dst_v, rows_v, acc, sem):
        cid = lax.axis_index("c")
        sid = lax.axis_index("s")
        pltpu.sync_copy(zeros_hbm, acc.at[pl.ds(sid * ROWS_A, ROWS_A)])

        @pl.when(sid == 0)
        def _zero_tail():
            pltpu.sync_copy(zeros_hbm.at[pl.ds(0, ROWS_TAIL)],
                            acc.at[pl.ds(NTILES * ROWS_A, ROWS_TAIL)])

        plsc.subcore_barrier()
        wid = cid * NTILES + sid

        def body(i, carry):
            base = wid * E_PER_TILE + i * K
            pltpu.sync_copy(fidx_hbm.at[pl.ds(base, K)], idx_v)
            pltpu.sync_copy(dst_hbm.at[pl.ds(base, K)], dst_v)
            pltpu.async_copy(table_hbm.at[idx_v], rows_v, sem).wait()
            pltpu.sync_copy(rows_v, acc.at[dst_v], add=True)
            return carry

        lax.fori_loop(0, N_CHUNKS, body, 0)
        plsc.subcore_barrier()
        pltpu.sync_copy(acc.at[pl.ds(sid * ROWS_A, ROWS_A)],
                        out_hbm.at[cid].at[pl.ds(sid * ROWS_A, ROWS_A)])

        @pl.when(sid == 0)
        def _copy_tail():
            pltpu.sync_copy(acc.at[pl.ds(NTILES * ROWS_A, ROWS_TAIL)],
                            out_hbm.at[cid].at[pl.ds(NTILES * ROWS_A,
                                                     ROWS_TAIL)])

    return k(table, fidx, dst, zeros)


def kernel(annotation, edge_index, etypes, W_e, b_e, w_ih, w_hh, b_ih, b_hh,
           w_out, b_out):
    ann = annotation.astype(jnp.float32)
    src = edge_index[0].astype(jnp.int32)
    dst = edge_index[1].astype(jnp.int32)
    et = etypes.astype(jnp.int32)
    fidx = src * N_ETYPES + et

    W_all = W_e.transpose(2, 0, 1).reshape(OUT, N_ETYPES * OUT)
    b_all = b_e.reshape(1, N_ETYPES * OUT)
    wihT = w_ih.T
    whhT = w_hh.T
    bih = b_ih.reshape(1, 3 * OUT)
    bhh = b_hh.reshape(1, 3 * OUT)
    w_h = w_out[:OUT, 0].reshape(1, OUT)
    w_a = w_out[OUT:, 0].reshape(1, ANN)
    b_o = b_out.reshape(1, 1)
    zeros = jnp.zeros((ROWS_A, OUT), jnp.float32)

    h = jnp.concatenate([ann, jnp.zeros((N, OUT - ANN), jnp.float32)], axis=1)
    tbl = _transform(h, W_all, b_all)
    for step in range(N_STEPS):
        ap = _sc_scatter(tbl.reshape(N_ETYPES * N, OUT), fidx, dst, zeros)
        if step < N_STEPS - 1:
            h, tbl = _gru_transform(ap[0], ap[1], h, wihT, whhT, bih, bhh,
                                    W_all, b_all)
        else:
            logits2d = _gru_final(ap[0], ap[1], h, wihT, whhT, bih, bhh,
                                  ann, w_h, w_a, b_o)

    logits = logits2d.reshape(N)
    padded = jnp.concatenate(
        [logits, jnp.full((NPAD - N,), -jnp.inf, jnp.float32)]).reshape(1, NPAD)
    pred = _argmax(padded).reshape(())
    return logits, pred


# speedup vs baseline: 34.7228x; 2.0201x over previous
import functools

import jax
import jax.numpy as jnp
from jax import lax
from jax.experimental import pallas as pl
from jax.experimental.pallas import tpu as pltpu
from jax.experimental.pallas import tpu_sc as plsc

N = 10000
E = 320000
ANN = 64
OUT = 128
N_STEPS = 5
N_ETYPES = 4

NSC = 2
NTILES = 16
NW = NSC * NTILES
ROWS_A = 624
ROWS_TAIL = N - NTILES * ROWS_A
K = 128
N_CHUNKS = E // K
CHUNKS_PER_TILE = N_CHUNKS // NW
CHUNKS_EXTRA = N_CHUNKS - NW * CHUNKS_PER_TILE

BM = 2000


def _transform_body(h_ref, w_ref, b_ref, t_ref):
    t_ref[...] = jnp.dot(h_ref[...], w_ref[...],
                         preferred_element_type=jnp.float32) + b_ref[...]


def _transform(h, W_all, b_all):
    return pl.pallas_call(
        _transform_body,
        grid=(N // BM,),
        in_specs=[pl.BlockSpec((BM, OUT), lambda i: (i, 0)),
                  pl.BlockSpec((OUT, 4 * OUT), lambda i: (0, 0)),
                  pl.BlockSpec((1, 4 * OUT), lambda i: (0, 0))],
        out_specs=pl.BlockSpec((BM, 4 * OUT), lambda i: (i, 0)),
        out_shape=jax.ShapeDtypeStruct((N, 4 * OUT), jnp.float32),
    )(h, W_all, b_all)


def _gru_math(a0_ref, a1_ref, h_ref, wih_ref, whh_ref, bih_ref, bhh_ref):
    a = a0_ref[...] + a1_ref[...]
    h = h_ref[...]
    gi = jnp.dot(a, wih_ref[...], preferred_element_type=jnp.float32) + bih_ref[...]
    gh = jnp.dot(h, whh_ref[...], preferred_element_type=jnp.float32) + bhh_ref[...]
    r = jax.nn.sigmoid(gi[:, :OUT] + gh[:, :OUT])
    z = jax.nn.sigmoid(gi[:, OUT:2 * OUT] + gh[:, OUT:2 * OUT])
    n = jnp.tanh(gi[:, 2 * OUT:] + r * gh[:, 2 * OUT:])
    return (1.0 - z) * n + z * h


def _gru_tr_body(a0_ref, a1_ref, h_ref, wih_ref, whh_ref, bih_ref, bhh_ref,
                 wall_ref, ball_ref, h_out_ref, t_out_ref):
    hn = _gru_math(a0_ref, a1_ref, h_ref, wih_ref, whh_ref, bih_ref, bhh_ref)
    h_out_ref[...] = hn
    t_out_ref[...] = jnp.dot(hn, wall_ref[...],
                             preferred_element_type=jnp.float32) + ball_ref[...]


def _gru_transform(a0, a1, h, wihT, whhT, bih, bhh, W_all, b_all):
    return pl.pallas_call(
        _gru_tr_body,
        grid=(N // BM,),
        in_specs=[pl.BlockSpec((BM, OUT), lambda i: (i, 0)),
                  pl.BlockSpec((BM, OUT), lambda i: (i, 0)),
                  pl.BlockSpec((BM, OUT), lambda i: (i, 0)),
                  pl.BlockSpec((OUT, 3 * OUT), lambda i: (0, 0)),
                  pl.BlockSpec((OUT, 3 * OUT), lambda i: (0, 0)),
                  pl.BlockSpec((1, 3 * OUT), lambda i: (0, 0)),
                  pl.BlockSpec((1, 3 * OUT), lambda i: (0, 0)),
                  pl.BlockSpec((OUT, 4 * OUT), lambda i: (0, 0)),
                  pl.BlockSpec((1, 4 * OUT), lambda i: (0, 0))],
        out_specs=[pl.BlockSpec((BM, OUT), lambda i: (i, 0)),
                   pl.BlockSpec((BM, 4 * OUT), lambda i: (i, 0))],
        out_shape=[jax.ShapeDtypeStruct((N, OUT), jnp.float32),
                   jax.ShapeDtypeStruct((N, 4 * OUT), jnp.float32)],
    )(a0, a1, h, wihT, whhT, bih, bhh, W_all, b_all)


def _gru_fin_body(a0_ref, a1_ref, h_ref, wih_ref, whh_ref, bih_ref, bhh_ref,
                  ann_ref, wh_ref, wa_ref, bo_ref, l_out_ref):
    hn = _gru_math(a0_ref, a1_ref, h_ref, wih_ref, whh_ref, bih_ref, bhh_ref)
    logit = (jnp.sum(hn * wh_ref[...], axis=1, keepdims=True)
             + jnp.sum(ann_ref[...] * wa_ref[...], axis=1, keepdims=True)
             + bo_ref[...])
    l_out_ref[...] = logit


def _gru_final(a0, a1, h, wihT, whhT, bih, bhh, ann, w_h, w_a, b_o):
    return pl.pallas_call(
        _gru_fin_body,
        grid=(N // BM,),
        in_specs=[pl.BlockSpec((BM, OUT), lambda i: (i, 0)),
                  pl.BlockSpec((BM, OUT), lambda i: (i, 0)),
                  pl.BlockSpec((BM, OUT), lambda i: (i, 0)),
                  pl.BlockSpec((OUT, 3 * OUT), lambda i: (0, 0)),
                  pl.BlockSpec((OUT, 3 * OUT), lambda i: (0, 0)),
                  pl.BlockSpec((1, 3 * OUT), lambda i: (0, 0)),
                  pl.BlockSpec((1, 3 * OUT), lambda i: (0, 0)),
                  pl.BlockSpec((BM, ANN), lambda i: (i, 0)),
                  pl.BlockSpec((1, OUT), lambda i: (0, 0)),
                  pl.BlockSpec((1, ANN), lambda i: (0, 0)),
                  pl.BlockSpec((1, 1), lambda i: (0, 0))],
        out_specs=pl.BlockSpec((BM, 1), lambda i: (i, 0)),
        out_shape=jax.ShapeDtypeStruct((N, 1), jnp.float32),
    )(a0, a1, h, wihT, whhT, bih, bhh, ann, w_h, w_a, b_o)


NPAD = 10240


def _argmax_body(l_ref, p_ref):
    row = l_ref[...]
    m = jnp.max(row)
    ii = lax.broadcasted_iota(jnp.int32, row.shape, 1)
    cand = jnp.where(row == m, ii, jnp.int32(NPAD))
    p_ref[...] = jnp.min(cand)[None, None]


def _argmax(logits_padded):
    return pl.pallas_call(
        _argmax_body,
        out_shape=jax.ShapeDtypeStruct((1, 1), jnp.int32),
    )(logits_padded)


def _sc_scatter(table, packed, zeros):
    mesh = plsc.VectorSubcoreMesh(core_axis_name="c", subcore_axis_name="s",
                                  num_cores=NSC, num_subcores=NTILES)

    @functools.partial(
        pl.kernel,
        out_type=jax.ShapeDtypeStruct((NSC, N, OUT), jnp.float32),
        mesh=mesh,
        scratch_types=[
            pltpu.VMEM((2, K), jnp.int32),
            pltpu.VMEM((2, K), jnp.int32),
            pltpu.VMEM((K, OUT), jnp.float32),
            pltpu.VMEM((K, OUT), jnp.float32),
            pltpu.VMEM_SHARED((N, OUT), jnp.float32),
            pltpu.SemaphoreType.DMA,
            pltpu.SemaphoreType.DMA,
        ],
    )
    def k(table_hbm, packed_hbm, zeros_hbm, out_hbm,
          idx_a, idx_b, buf_a, buf_b, acc, sem_a, sem_b):
        cid = lax.axis_index("c")
        sid = lax.axis_index("s")
        pltpu.sync_copy(zeros_hbm, acc.at[pl.ds(sid * ROWS_A, ROWS_A)])

        @pl.when(sid == 0)
        def _zero_tail():
            pltpu.sync_copy(zeros_hbm.at[pl.ds(0, ROWS_TAIL)],
                            acc.at[pl.ds(NTILES * ROWS_A, ROWS_TAIL)])

        plsc.subcore_barrier()
        wid = cid * NTILES + sid
        cbase = wid * CHUNKS_PER_TILE

        def start(idx_v, buf, sem, chunk):
            pltpu.sync_copy(packed_hbm.at[chunk], idx_v)
            return pltpu.async_copy(table_hbm.at[idx_v.at[0]], buf, sem)

        def drain_scatter(idx_v, buf, sem):
            pltpu.make_async_copy(table_hbm.at[idx_v.at[0]], buf, sem).wait()
            pltpu.sync_copy(buf, acc.at[idx_v.at[1]], add=True)

        start(idx_a, buf_a, sem_a, cbase)

        def body(j, carry):
            a = cbase + 2 * j
            start(idx_b, buf_b, sem_b, a + 1)
            drain_scatter(idx_a, buf_a, sem_a)

            @pl.when(j < CHUNKS_PER_TILE // 2 - 1)
            def _next():
                start(idx_a, buf_a, sem_a, a + 2)

            drain_scatter(idx_b, buf_b, sem_b)
            return carry

        lax.fori_loop(0, CHUNKS_PER_TILE // 2, body, 0)

        @pl.when(wid < CHUNKS_EXTRA)
        def _extra():
            start(idx_a, buf_a, sem_a, NW * CHUNKS_PER_TILE + wid).wait()
            pltpu.sync_copy(buf_a, acc.at[idx_a.at[1]], add=True)

        plsc.subcore_barrier()
        pltpu.sync_copy(acc.at[pl.ds(sid * ROWS_A, ROWS_A)],
                        out_hbm.at[cid].at[pl.ds(sid * ROWS_A, ROWS_A)])

        @pl.when(sid == 0)
        def _copy_tail():
            pltpu.sync_copy(acc.at[pl.ds(NTILES * ROWS_A, ROWS_TAIL)],
                            out_hbm.at[cid].at[pl.ds(NTILES * ROWS_A,
                                                     ROWS_TAIL)])

    return k(table, packed, zeros)


def kernel(annotation, edge_index, etypes, W_e, b_e, w_ih, w_hh, b_ih, b_hh,
           w_out, b_out):
    ann = annotation.astype(jnp.float32)
    src = edge_index[0].astype(jnp.int32)
    dst = edge_index[1].astype(jnp.int32)
    et = etypes.astype(jnp.int32)
    fidx = src * N_ETYPES + et
    packed = jnp.stack([fidx.reshape(N_CHUNKS, K), dst.reshape(N_CHUNKS, K)],
                       axis=1)

    W_all = W_e.transpose(2, 0, 1).reshape(OUT, N_ETYPES * OUT)
    b_all = b_e.reshape(1, N_ETYPES * OUT)
    wihT = w_ih.T
    whhT = w_hh.T
    bih = b_ih.reshape(1, 3 * OUT)
    bhh = b_hh.reshape(1, 3 * OUT)
    w_h = w_out[:OUT, 0].reshape(1, OUT)
    w_a = w_out[OUT:, 0].reshape(1, ANN)
    b_o = b_out.reshape(1, 1)
    zeros = jnp.zeros((ROWS_A, OUT), jnp.float32)

    h = jnp.concatenate([ann, jnp.zeros((N, OUT - ANN), jnp.float32)], axis=1)
    tbl = _transform(h, W_all, b_all)
    for step in range(N_STEPS):
        ap = _sc_scatter(tbl.reshape(N_ETYPES * N, OUT), packed, zeros)
        if step < N_STEPS - 1:
            h, tbl = _gru_transform(ap[0], ap[1], h, wihT, whhT, bih, bhh,
                                    W_all, b_all)
        else:
            logits2d = _gru_final(ap[0], ap[1], h, wihT, whhT, bih, bhh,
                                  ann, w_h, w_a, b_o)

    logits = logits2d.reshape(N)
    padded = jnp.concatenate(
        [logits, jnp.full((NPAD - N,), -jnp.inf, jnp.float32)]).reshape(1, NPAD)
    pred = _argmax(padded).reshape(())
    return logits, pred
